# baseline (speedup 1.0000x reference)
import functools
import math

import jax
import jax.numpy as jnp
from jax import lax
from jax.experimental import pallas as pl
from jax.experimental.pallas import tpu as pltpu
from jax.experimental.pallas import tpu_sc as plsc

D_MODEL = 64
SCALE = math.sqrt(D_MODEL)
IDXW = 128
GPC = 4
CHUNK = IDXW * GPC


def _emb_body(idx_hbm, lut_hbm, out_hbm, idx_v, rows_v, sem, *, rows_per_w):
    nc = plsc.get_sparse_core_info().num_cores
    wid = lax.axis_index("s") * nc + lax.axis_index("c")
    irows_per_w = rows_per_w // IDXW
    chunks = rows_per_w // CHUNK
    ibase = wid * irows_per_w

    def chunk_body(g, _):
        irow = ibase + g * GPC
        pltpu.sync_copy(idx_hbm.at[pl.ds(irow, GPC)], idx_v)
        copies = [
            pltpu.async_copy(
                lut_hbm.at[idx_v.at[j]],
                rows_v.at[pl.ds(j * IDXW, IDXW)],
                sem,
            )
            for j in range(GPC)
        ]
        for c in copies:
            c.wait()

        def scale_body(i, _):
            for j in range(D_MODEL // 16):
                rows_v[i, pl.ds(j * 16, 16)] = (
                    rows_v[i, pl.ds(j * 16, 16)] * SCALE
                )
            return ()

        lax.fori_loop(0, CHUNK, scale_body, ())
        pltpu.sync_copy(rows_v, out_hbm.at[pl.ds(irow * IDXW, CHUNK)])
        return ()

    lax.fori_loop(0, chunks, chunk_body, ())


def kernel(input_data, lut):
    s0, s1 = input_data.shape
    b = s0 * s1
    info = plsc.get_sparse_core_info()
    nw = info.num_cores * info.num_subcores
    rows_per_w = b // nw
    assert rows_per_w % CHUNK == 0

    idx2d = input_data.reshape(b // IDXW, IDXW).astype(jnp.int32)

    mesh = plsc.VectorSubcoreMesh(core_axis_name="c", subcore_axis_name="s")
    emb = functools.partial(
        pl.kernel,
        mesh=mesh,
        out_type=jax.ShapeDtypeStruct((b, D_MODEL), jnp.float32),
        scratch_types=[
            pltpu.VMEM((GPC, IDXW), jnp.int32),
            pltpu.VMEM((CHUNK, D_MODEL), jnp.float32),
            pltpu.SemaphoreType.DMA,
        ],
        compiler_params=pltpu.CompilerParams(use_tc_tiling_on_sc=False),
    )(functools.partial(_emb_body, rows_per_w=rows_per_w))

    out = emb(idx2d, lut)
    return out.reshape(s0, s1, D_MODEL)


# speedup vs baseline: 1.1339x; 1.1339x over previous
import functools
import math

import jax
import jax.numpy as jnp
from jax import lax
from jax.experimental import pallas as pl
from jax.experimental.pallas import tpu as pltpu
from jax.experimental.pallas import tpu_sc as plsc

D_MODEL = 64
SCALE = math.sqrt(D_MODEL)
IDXW = 128
GPC = 4
CHUNK = IDXW * GPC
UNROLL = 8


def _emb_body(idx_hbm, lut_hbm, out_hbm, idx_v, rows_v,
              gsem0, gsem1, osem0, osem1, *, rows_per_w):
    nc = plsc.get_sparse_core_info().num_cores
    wid = lax.axis_index("s") * nc + lax.axis_index("c")
    irows_per_w = rows_per_w // IDXW
    chunks = rows_per_w // CHUNK
    ibase = wid * irows_per_w
    obase = wid * rows_per_w
    gsems = (gsem0, gsem1)
    osems = (osem0, osem1)

    pltpu.sync_copy(idx_hbm.at[pl.ds(ibase, irows_per_w)], idx_v)

    def fire_gather(g, b):
        for j in range(GPC):
            pltpu.async_copy(
                lut_hbm.at[idx_v.at[g * GPC + j]],
                rows_v.at[b].at[pl.ds(j * IDXW, IDXW)],
                gsems[b],
            )

    def wait_gather(b):
        for _ in range(GPC):
            pltpu.make_async_copy(
                lut_hbm.at[idx_v.at[0]],
                rows_v.at[b].at[pl.ds(0, IDXW)],
                gsems[b],
            ).wait()

    def scale(b):
        @pl.loop(0, CHUNK, step=UNROLL)
        def _(i):
            for r in range(UNROLL):
                for j in range(D_MODEL // 16):
                    rows_v[b, i + r, pl.ds(j * 16, 16)] = (
                        rows_v[b, i + r, pl.ds(j * 16, 16)] * SCALE
                    )

    def fire_store(g, b):
        pltpu.async_copy(
            rows_v.at[b],
            out_hbm.at[pl.ds(obase + g * CHUNK, CHUNK)],
            osems[b],
        )

    def wait_store(b):
        pltpu.make_async_copy(
            rows_v.at[b],
            out_hbm.at[pl.ds(obase, CHUNK)],
            osems[b],
        ).wait()

    fire_gather(0, 0)
    fire_gather(1, 1)
    wait_gather(0)
    scale(0)
    fire_store(0, 0)

    @pl.loop(1, chunks - 1, step=2)
    def _(go):
        for db in range(2):
            g = go + db
            b = (1 + db) % 2
            nb = 1 - b
            wait_store(nb)
            fire_gather(g + 1, nb)
            wait_gather(b)
            scale(b)
            fire_store(g, b)

    wait_gather(1)
    scale(1)
    fire_store(chunks - 1, 1)
    wait_store(0)
    wait_store(1)


def kernel(input_data, lut):
    s0, s1 = input_data.shape
    b = s0 * s1
    info = plsc.get_sparse_core_info()
    nw = info.num_cores * info.num_subcores
    rows_per_w = b // nw
    assert rows_per_w % CHUNK == 0 and (rows_per_w // CHUNK) % 2 == 0

    idx2d = input_data.reshape(b // IDXW, IDXW).astype(jnp.int32)

    mesh = plsc.VectorSubcoreMesh(core_axis_name="c", subcore_axis_name="s")
    emb = functools.partial(
        pl.kernel,
        mesh=mesh,
        out_type=jax.ShapeDtypeStruct((b, D_MODEL), jnp.float32),
        scratch_types=[
            pltpu.VMEM((rows_per_w // IDXW, IDXW), jnp.int32),
            pltpu.VMEM((2, CHUNK, D_MODEL), jnp.float32),
            pltpu.SemaphoreType.DMA,
            pltpu.SemaphoreType.DMA,
            pltpu.SemaphoreType.DMA,
            pltpu.SemaphoreType.DMA,
        ],
        compiler_params=pltpu.CompilerParams(use_tc_tiling_on_sc=False),
    )(functools.partial(_emb_body, rows_per_w=rows_per_w))

    out = emb(idx2d, lut)
    return out.reshape(s0, s1, D_MODEL)
